# baseline (speedup 1.0000x reference)
import functools

import jax
import jax.numpy as jnp
import numpy as np
from jax import lax
from jax.experimental import pallas as pl
from jax.experimental.pallas import tpu as pltpu
from jax.experimental.pallas import tpu_sc as plsc

D = 256
H = 8
DH = D // H
N_AGENT = 256
N_MAP = 2048
KNN = 20
D_FF = 2048


def _pe_consts():
    col = jax.lax.broadcasted_iota(jnp.int32, (1, D), 1)
    seg = col // 64
    cmod = (col % 64) % 32
    logt = jnp.where(seg < 2, np.float32(np.log(1000.0)),
                     np.float32(np.log(10.0)))
    fr = jnp.exp(cmod.astype(jnp.float32) * (-2.0 / 64.0) * logt)
    phase = jnp.where((col % 64) < 32, np.float32(np.pi / 2),
                      np.float32(0.0))
    comp = jax.lax.broadcasted_iota(jnp.int32, (8, D), 0)
    sel = jnp.where((comp == seg) & (comp < 4), fr, np.float32(0.0))
    return sel, phase


def _seg_mask():
    d = jax.lax.broadcasted_iota(jnp.int32, (D, H), 0)
    h = jax.lax.broadcasted_iota(jnp.int32, (D, H), 1)
    return (d // DH == h).astype(jnp.float32)


def _dense_body(actors_ref, sc_ref, Wrpe_ref, Wq_ref,
                Wk_ref, Wv_ref, Wo_ref, ln1g_ref, ln1b_ref, Wf1_ref,
                bf1_ref, Wf2_ref, bf2_ref, ln2g_ref, ln2b_ref, out_ref):
    f32 = jnp.float32
    actors = actors_ref[...]
    sc = sc_ref[...]
    x0 = sc[:, 0:KNN]
    x1 = sc[:, 32:32 + KNN]
    th = sc[:, 64:64 + KNN]
    blk = x0.shape[0]

    sel, phase = _pe_consts()
    seg = _seg_mask()

    comps = jnp.concatenate(
        [x0[..., None], x1[..., None], jnp.cos(th)[..., None],
         jnp.sin(th)[..., None], jnp.zeros((blk, KNN, 4), f32)],
        axis=-1)
    rpe2 = jnp.sin(comps.reshape(blk * KNN, 8) @ sel + phase)

    bf16 = jnp.bfloat16

    def mm16(a, b):
        return jax.lax.dot(a.astype(bf16), b.astype(bf16),
                           preferred_element_type=f32)

    Wrk = Wrpe_ref[...] @ Wk_ref[...]
    Wrv = Wrpe_ref[...] @ Wv_ref[...]
    Rk2 = mm16(rpe2, Wrk)
    Rv2 = mm16(rpe2, Wrv)
    q = actors @ Wq_ref[...]
    qb = jnp.broadcast_to(q[:, None, :], (blk, KNN, D)).reshape(blk * KNN, D)
    logits = ((qb * Rk2) @ seg) * (1.0 / np.sqrt(DH))
    p8 = jnp.exp(logits)
    pe256 = p8 @ seg.T
    row = jax.lax.broadcasted_iota(jnp.int32, (blk, blk * KNN), 0)
    col = jax.lax.broadcasted_iota(jnp.int32, (blk, blk * KNN), 1)
    s2 = (col // KNN == row).astype(f32)
    num = s2 @ (pe256 * Rv2)
    den = s2 @ pe256
    ctx = num / den + actors @ Wv_ref[...]

    def ln(x, g, b):
        mu = jnp.mean(x, axis=-1, keepdims=True)
        var = jnp.mean((x - mu) ** 2, axis=-1, keepdims=True)
        return (x - mu) / jnp.sqrt(var + 1e-5) * g + b

    x = ln(actors + ctx @ Wo_ref[...], ln1g_ref[...], ln1b_ref[...])
    ff = mm16(jnp.maximum(mm16(x, Wf1_ref[...]) + bf1_ref[...], 0.0),
              Wf2_ref[...])
    ff = ff + bf2_ref[...]
    out_ref[...] = ln(x + ff, ln2g_ref[...], ln2b_ref[...])


_BLK = 64


def _fixed(shape):
    return pl.BlockSpec(shape, lambda i: tuple(0 for _ in shape))


@jax.jit
def _dense_block(actors, sc_out, W_rpe, Wq, Wk, Wv, Wo, ln1_g, ln1_b,
                 W_ff1, b_ff1, W_ff2, b_ff2, ln2_g, ln2_b):
    nblk = N_AGENT // _BLK
    row_spec = pl.BlockSpec((_BLK, D), lambda i: (i, 0))
    sc_spec = pl.BlockSpec((_BLK, 96), lambda i: (i, 0))
    return pl.pallas_call(
        _dense_body,
        grid=(nblk,),
        in_specs=[row_spec, sc_spec,
                  _fixed((D, D)), _fixed((D, D)), _fixed((D, D)),
                  _fixed((D, D)), _fixed((D, D)),
                  _fixed((1, D)), _fixed((1, D)),
                  _fixed((D, D_FF)), _fixed((1, D_FF)),
                  _fixed((D_FF, D)), _fixed((1, D)),
                  _fixed((1, D)), _fixed((1, D))],
        out_specs=row_spec,
        out_shape=jax.ShapeDtypeStruct((N_AGENT, D), jnp.float32),
    )(actors, sc_out, W_rpe, Wq, Wk, Wv, Wo,
      ln1_g.reshape(1, D), ln1_b.reshape(1, D),
      W_ff1, b_ff1.reshape(1, D_FF), W_ff2, b_ff2.reshape(1, D),
      ln2_g.reshape(1, D), ln2_b.reshape(1, D))


N_ALL = N_AGENT + N_MAP
_ROWS_PER_W = N_AGENT // 32


def _sc_topk_body(rd_hbm, rp_hbm, out_hbm, rows_v, rps_v, v0, v1, v2, v3):
    info = plsc.get_sparse_core_info()
    nc = info.num_cores
    wid = lax.axis_index("s") * nc + lax.axis_index("c")
    f32 = jnp.float32
    i32 = jnp.int32
    inf16 = jnp.full((16,), jnp.inf, f32)
    zero16 = jnp.zeros((16,), i32)
    lane = lax.iota(i32, 16)

    def merge(src_off, j, b0k, b0v, b1k, b1v):
        ck = rows_v[pl.ds(src_off + j * 16, 16)]
        cv = lane + j * 16
        ck, cv = plsc.sort_key_val(ck, cv)
        rck = lax.rev(ck, (0,))
        rcv = lax.rev(cv, (0,))
        m1 = b1k <= rck
        lk = jnp.where(m1, b1k, rck)
        lv = jnp.where(m1, b1v, rcv)
        lk, lv = plsc.sort_key_val(lk, lv)
        rlk = lax.rev(lk, (0,))
        rlv = lax.rev(lv, (0,))
        m2 = b0k <= rlk
        nb0k = jnp.where(m2, b0k, rlk)
        nb0v = jnp.where(m2, b0v, rlv)
        nb1k = jnp.where(m2, rlk, b0k)
        nb1v = jnp.where(m2, rlv, b0v)
        b0k, b0v = plsc.sort_key_val(nb0k, nb0v)
        b1k, b1v = plsc.sort_key_val(nb1k, nb1v)
        return b0k, b0v, b1k, b1v

    nlace = 4

    def chunk(j, carry):
        return tuple(merge(i * N_MAP, j, *carry[i]) for i in range(nlace))

    vrefs = (v0, v1, v2, v3)

    def emit(i, b0v, b1v, row):
        vals = vrefs[i]
        for c in range(3):
            csplat = jnp.full((16,), c + i * N_MAP * 3, i32)
            vals[pl.ds(c * 32, 16)] = plsc.load_gather(
                rps_v, [b0v * 3 + csplat])
            vals[pl.ds(c * 32 + 16, 16)] = plsc.load_gather(
                rps_v, [b1v * 3 + csplat])
        pltpu.sync_copy(vals, out_hbm.at[row])

    def do_quad(r, _):
        base_row = wid * _ROWS_PER_W + nlace * r
        for i in range(nlace):
            pltpu.sync_copy(rd_hbm.at[base_row + i],
                            rows_v.at[pl.ds(i * N_MAP, N_MAP)])
            pltpu.sync_copy(rp_hbm.at[base_row + i],
                            rps_v.at[pl.ds(i * N_MAP * 3, N_MAP * 3)])
        init = (inf16, zero16, inf16, zero16)
        res = lax.fori_loop(0, N_MAP // 16, chunk, (init,) * nlace)
        for i in range(nlace):
            emit(i, res[i][1], res[i][3], base_row + i)
        return 0

    lax.fori_loop(0, _ROWS_PER_W // nlace, do_quad, 0)


@jax.jit
def _sc_topk(rd2, rp2):
    fn = functools.partial(
        pl.kernel,
        mesh=plsc.VectorSubcoreMesh(core_axis_name="c", subcore_axis_name="s"),
        out_type=jax.ShapeDtypeStruct((N_AGENT, 96), jnp.float32),
        scratch_types=[
            pltpu.VMEM((4 * N_MAP,), jnp.float32),
            pltpu.VMEM((4 * N_MAP * 3,), jnp.float32),
            pltpu.VMEM((96,), jnp.float32),
            pltpu.VMEM((96,), jnp.float32),
            pltpu.VMEM((96,), jnp.float32),
            pltpu.VMEM((96,), jnp.float32),
        ],
        compiler_params=pltpu.CompilerParams(needs_layout_passes=False),
    )(_sc_topk_body)
    return fn(rd2, rp2)


def kernel(actors, actor_idcs, lanes, lane_idcs, rpe_scene, rel_pose,
           W_rpe, Wq, Wk, Wv, Wo, ln1_g, ln1_b, W_ff1, b_ff1, W_ff2,
           b_ff2, ln2_g, ln2_b):
    rd2 = rpe_scene[2, :N_AGENT, N_AGENT:]
    rp2 = rel_pose[:N_AGENT, N_AGENT:, :].reshape(N_AGENT, N_MAP * 3)
    sc_out = _sc_topk(rd2, rp2)
    x = _dense_block(actors, sc_out,
                     W_rpe, Wq, Wk, Wv, Wo, ln1_g, ln1_b,
                     W_ff1, b_ff1, W_ff2, b_ff2, ln2_g, ln2_b)
    return (x, lanes)


# speedup vs baseline: 1.0329x; 1.0329x over previous
import functools

import jax
import jax.numpy as jnp
import numpy as np
from jax import lax
from jax.experimental import pallas as pl
from jax.experimental.pallas import tpu as pltpu
from jax.experimental.pallas import tpu_sc as plsc

D = 256
H = 8
DH = D // H
N_AGENT = 256
N_MAP = 2048
KNN = 20
D_FF = 2048


def _pe_consts():
    col = jax.lax.broadcasted_iota(jnp.int32, (1, D), 1)
    seg = col // 64
    cmod = (col % 64) % 32
    logt = jnp.where(seg < 2, np.float32(np.log(1000.0)),
                     np.float32(np.log(10.0)))
    fr = jnp.exp(cmod.astype(jnp.float32) * (-2.0 / 64.0) * logt)
    phase = jnp.where((col % 64) < 32, np.float32(np.pi / 2),
                      np.float32(0.0))
    comp = jax.lax.broadcasted_iota(jnp.int32, (8, D), 0)
    sel = jnp.where((comp == seg) & (comp < 4), fr, np.float32(0.0))
    return sel, phase


def _seg_mask():
    d = jax.lax.broadcasted_iota(jnp.int32, (D, H), 0)
    h = jax.lax.broadcasted_iota(jnp.int32, (D, H), 1)
    return (d // DH == h).astype(jnp.float32)


def _dense_body(actors_ref, sc_ref, Wrpe_ref, Wq_ref,
                Wk_ref, Wv_ref, Wo_ref, ln1g_ref, ln1b_ref, Wf1_ref,
                bf1_ref, Wf2_ref, bf2_ref, ln2g_ref, ln2b_ref, out_ref):
    f32 = jnp.float32
    actors = actors_ref[...]
    sc = sc_ref[...]
    x0 = sc[:, 0:KNN]
    x1 = sc[:, 32:32 + KNN]
    th = sc[:, 64:64 + KNN]
    blk = x0.shape[0]

    sel, phase = _pe_consts()
    seg = _seg_mask()

    comps = jnp.concatenate(
        [x0[..., None], x1[..., None], jnp.cos(th)[..., None],
         jnp.sin(th)[..., None], jnp.zeros((blk, KNN, 4), f32)],
        axis=-1)
    rpe2 = jnp.sin(comps.reshape(blk * KNN, 8) @ sel + phase)

    Wrk = Wrpe_ref[...] @ Wk_ref[...]
    Wrv = Wrpe_ref[...] @ Wv_ref[...]
    Rk2 = rpe2 @ Wrk
    Rv2 = rpe2 @ Wrv
    q = actors @ Wq_ref[...]
    qb = jnp.broadcast_to(q[:, None, :], (blk, KNN, D)).reshape(blk * KNN, D)
    logits = ((qb * Rk2) @ seg) * (1.0 / np.sqrt(DH))
    p8 = jnp.exp(logits)
    pe256 = p8 @ seg.T
    row = jax.lax.broadcasted_iota(jnp.int32, (blk, blk * KNN), 0)
    col = jax.lax.broadcasted_iota(jnp.int32, (blk, blk * KNN), 1)
    s2 = (col // KNN == row).astype(f32)
    num = s2 @ (pe256 * Rv2)
    den = s2 @ pe256
    ctx = num / den + actors @ Wv_ref[...]

    def ln(x, g, b):
        mu = jnp.mean(x, axis=-1, keepdims=True)
        var = jnp.mean((x - mu) ** 2, axis=-1, keepdims=True)
        return (x - mu) / jnp.sqrt(var + 1e-5) * g + b

    x = ln(actors + ctx @ Wo_ref[...], ln1g_ref[...], ln1b_ref[...])
    ff = jnp.maximum(x @ Wf1_ref[...] + bf1_ref[...], 0.0) @ Wf2_ref[...]
    ff = ff + bf2_ref[...]
    out_ref[...] = ln(x + ff, ln2g_ref[...], ln2b_ref[...])


_BLK = 128


def _fixed(shape):
    return pl.BlockSpec(shape, lambda i: tuple(0 for _ in shape))


@jax.jit
def _dense_block(actors, sc_out, W_rpe, Wq, Wk, Wv, Wo, ln1_g, ln1_b,
                 W_ff1, b_ff1, W_ff2, b_ff2, ln2_g, ln2_b):
    nblk = N_AGENT // _BLK
    row_spec = pl.BlockSpec((_BLK, D), lambda i: (i, 0))
    sc_spec = pl.BlockSpec((_BLK, 96), lambda i: (i, 0))
    return pl.pallas_call(
        _dense_body,
        grid=(nblk,),
        in_specs=[row_spec, sc_spec,
                  _fixed((D, D)), _fixed((D, D)), _fixed((D, D)),
                  _fixed((D, D)), _fixed((D, D)),
                  _fixed((1, D)), _fixed((1, D)),
                  _fixed((D, D_FF)), _fixed((1, D_FF)),
                  _fixed((D_FF, D)), _fixed((1, D)),
                  _fixed((1, D)), _fixed((1, D))],
        out_specs=row_spec,
        out_shape=jax.ShapeDtypeStruct((N_AGENT, D), jnp.float32),
    )(actors, sc_out, W_rpe, Wq, Wk, Wv, Wo,
      ln1_g.reshape(1, D), ln1_b.reshape(1, D),
      W_ff1, b_ff1.reshape(1, D_FF), W_ff2, b_ff2.reshape(1, D),
      ln2_g.reshape(1, D), ln2_b.reshape(1, D))


N_ALL = N_AGENT + N_MAP
_ROWS_PER_W = N_AGENT // 32


def _sc_topk_body(rd_hbm, rp_hbm, out_hbm, rows_v, rps_v, v0, v1, v2, v3):
    info = plsc.get_sparse_core_info()
    nc = info.num_cores
    wid = lax.axis_index("s") * nc + lax.axis_index("c")
    f32 = jnp.float32
    i32 = jnp.int32
    inf16 = jnp.full((16,), jnp.inf, f32)
    zero16 = jnp.zeros((16,), i32)
    lane = lax.iota(i32, 16)

    def merge(src_off, j, b0k, b0v, b1k, b1v):
        ck = rows_v[pl.ds(src_off + j * 16, 16)]
        cv = lane + j * 16
        ck, cv = plsc.sort_key_val(ck, cv)
        rck = lax.rev(ck, (0,))
        rcv = lax.rev(cv, (0,))
        m1 = b1k <= rck
        lk = jnp.where(m1, b1k, rck)
        lv = jnp.where(m1, b1v, rcv)
        lk, lv = plsc.sort_key_val(lk, lv)
        rlk = lax.rev(lk, (0,))
        rlv = lax.rev(lv, (0,))
        m2 = b0k <= rlk
        nb0k = jnp.where(m2, b0k, rlk)
        nb0v = jnp.where(m2, b0v, rlv)
        nb1k = jnp.where(m2, rlk, b0k)
        nb1v = jnp.where(m2, rlv, b0v)
        b0k, b0v = plsc.sort_key_val(nb0k, nb0v)
        b1k, b1v = plsc.sort_key_val(nb1k, nb1v)
        return b0k, b0v, b1k, b1v

    nlace = 4

    def chunk(j, carry):
        return tuple(merge(i * N_MAP, j, *carry[i]) for i in range(nlace))

    vrefs = (v0, v1, v2, v3)

    def emit(i, b0v, b1v, row):
        vals = vrefs[i]
        for c in range(3):
            csplat = jnp.full((16,), c + i * N_MAP * 3, i32)
            vals[pl.ds(c * 32, 16)] = plsc.load_gather(
                rps_v, [b0v * 3 + csplat])
            vals[pl.ds(c * 32 + 16, 16)] = plsc.load_gather(
                rps_v, [b1v * 3 + csplat])
        pltpu.sync_copy(vals, out_hbm.at[row])

    def do_quad(r, _):
        base_row = wid * _ROWS_PER_W + nlace * r
        for i in range(nlace):
            pltpu.sync_copy(rd_hbm.at[base_row + i],
                            rows_v.at[pl.ds(i * N_MAP, N_MAP)])
            pltpu.sync_copy(rp_hbm.at[base_row + i],
                            rps_v.at[pl.ds(i * N_MAP * 3, N_MAP * 3)])
        init = (inf16, zero16, inf16, zero16)
        res = lax.fori_loop(0, N_MAP // 16, chunk, (init,) * nlace)
        for i in range(nlace):
            emit(i, res[i][1], res[i][3], base_row + i)
        return 0

    lax.fori_loop(0, _ROWS_PER_W // nlace, do_quad, 0)


@jax.jit
def _sc_topk(rd2, rp2):
    fn = functools.partial(
        pl.kernel,
        mesh=plsc.VectorSubcoreMesh(core_axis_name="c", subcore_axis_name="s"),
        out_type=jax.ShapeDtypeStruct((N_AGENT, 96), jnp.float32),
        scratch_types=[
            pltpu.VMEM((4 * N_MAP,), jnp.float32),
            pltpu.VMEM((4 * N_MAP * 3,), jnp.float32),
            pltpu.VMEM((96,), jnp.float32),
            pltpu.VMEM((96,), jnp.float32),
            pltpu.VMEM((96,), jnp.float32),
            pltpu.VMEM((96,), jnp.float32),
        ],
        compiler_params=pltpu.CompilerParams(needs_layout_passes=False),
    )(_sc_topk_body)
    return fn(rd2, rp2)


def kernel(actors, actor_idcs, lanes, lane_idcs, rpe_scene, rel_pose,
           W_rpe, Wq, Wk, Wv, Wo, ln1_g, ln1_b, W_ff1, b_ff1, W_ff2,
           b_ff2, ln2_g, ln2_b):
    rd2 = rpe_scene[2, :N_AGENT, N_AGENT:]
    rp2 = rel_pose[:N_AGENT, N_AGENT:, :].reshape(N_AGENT, N_MAP * 3)
    sc_out = _sc_topk(rd2, rp2)
    x = _dense_block(actors, sc_out,
                     W_rpe, Wq, Wk, Wv, Wo, ln1_g, ln1_b,
                     W_ff1, b_ff1, W_ff2, b_ff2, ln2_g, ln2_b)
    return (x, lanes)
